# baseline (speedup 1.0000x reference)
import functools
import math

import jax
import jax.numpy as jnp
from jax import lax
from jax.experimental import pallas as pl
from jax.experimental.pallas import tpu as pltpu
from jax.experimental.pallas import tpu_sc as plsc

EMBED = 64
SCALE = math.sqrt(EMBED)

NC = 2
NS = 16
NW = NC * NS
LANES = 16

A_BLK = 128
NBUF = 5
PREF = 3


def _build_lookup(b0, b1):
    assert b0 == NW * A_BLK
    n_chunk = b1
    assert n_chunk % NBUF == 0
    mesh = plsc.VectorSubcoreMesh(core_axis_name="c", subcore_axis_name="s")

    @functools.partial(
        pl.kernel,
        mesh=mesh,
        out_type=jax.ShapeDtypeStruct(
            (b1, EMBED // 8, NW, 8, A_BLK), jnp.float32
        ),
        compiler_params=pltpu.CompilerParams(
            use_tc_tiling_on_sc=False, needs_layout_passes=False
        ),
        scratch_types=(
            [pltpu.VMEM((n_chunk, A_BLK), jnp.int32)]
            + [pltpu.VMEM((A_BLK, EMBED), jnp.float32) for _ in range(NBUF)]
            + [pltpu.VMEM((8, 8, A_BLK), jnp.float32) for _ in range(NBUF)]
            + [pltpu.SemaphoreType.DMA for _ in range(2 * NBUF)]
        ),
    )
    def lookup(idx_hbm, table_hbm, out_hbm, idx_v, *scratch):
        gbufs = scratch[:NBUF]
        tbufs = scratch[NBUF : 2 * NBUF]
        gsem = scratch[2 * NBUF : 3 * NBUF]
        ssem = scratch[3 * NBUF :]

        wid = lax.axis_index("s") * NC + lax.axis_index("c")
        a0 = wid * A_BLK
        pltpu.sync_copy(idx_hbm.at[:, pl.ds(a0, A_BLK)], idx_v)

        row_idx = [
            lax.iota(jnp.int32, LANES) + LANES * j
            for j in range(A_BLK // LANES)
        ]

        def gather_start(c, b):
            pltpu.async_copy(table_hbm.at[idx_v.at[c]], gbufs[b], gsem[b])

        def gather_wait(c, b):
            pltpu.make_async_copy(
                table_hbm.at[idx_v.at[c]], gbufs[b], gsem[b]
            ).wait()

        def out_slice(c):
            return out_hbm.at[c, :, wid]

        def store_start(c, b):
            pltpu.async_copy(tbufs[b], out_slice(c), ssem[b])

        def store_wait(c, b):
            pltpu.make_async_copy(tbufs[b], out_slice(c), ssem[b]).wait()

        for c in range(PREF):
            gather_start(c, c)

        def outer(i, carry):
            for b in range(NBUF):
                c = i * NBUF + b
                gather_wait(c, b)

                gbuf, tbuf = gbufs[b], tbufs[b]

                @plsc.parallel_loop(0, EMBED, unroll=4)
                def _transpose_scale(cd):
                    col = jnp.full((LANES,), cd, jnp.int32)
                    tr = lax.shift_right_logical(cd, 3)
                    r = lax.bitwise_and(cd, 7)
                    for j in range(A_BLK // LANES):
                        v = plsc.load_gather(gbuf, [row_idx[j], col]) * SCALE
                        tbuf[tr, r, pl.ds(j * LANES, LANES)] = v

                store_start(c, b)

                bt = (b + PREF) % NBUF
                ct = c + PREF

                @pl.when(ct < n_chunk)
                def _prefetch():
                    @pl.when(c >= NBUF - PREF)
                    def _drain():
                        store_wait(ct - NBUF, bt)

                    gather_start(ct, bt)

            return carry

        lax.fori_loop(0, n_chunk // NBUF, outer, 0)

        for b in range(NBUF):
            store_wait(n_chunk - NBUF + b, b)

    return lookup


def kernel(inputs, table):
    b0, b1 = inputs.shape
    idx_t = jnp.transpose(inputs).astype(jnp.int32)
    out5 = _build_lookup(b0, b1)(idx_t, table)
    return out5.transpose(2, 4, 0, 1, 3).reshape(b0, b1, EMBED)


# speedup vs baseline: 1.6640x; 1.6640x over previous
import functools
import math

import jax
import jax.numpy as jnp
from jax import lax
from jax.experimental import pallas as pl
from jax.experimental.pallas import tpu as pltpu
from jax.experimental.pallas import tpu_sc as plsc

EMBED = 64
SCALE = math.sqrt(EMBED)

NC = 2
NS = 16
NW = NC * NS
LANES = 16

A_BLK = 128
NBUF = 5
PREF = 3


def _build_lookup(b0, b1):
    assert b0 == NW * A_BLK
    n_chunk = b1
    assert n_chunk % NBUF == 0
    mesh = plsc.VectorSubcoreMesh(core_axis_name="c", subcore_axis_name="s")

    @functools.partial(
        pl.kernel,
        mesh=mesh,
        out_type=jax.ShapeDtypeStruct(
            (b1, EMBED // 8, NW, 8, A_BLK), jnp.float32
        ),
        compiler_params=pltpu.CompilerParams(
            use_tc_tiling_on_sc=False, needs_layout_passes=False
        ),
        scratch_types=(
            [pltpu.VMEM((n_chunk, A_BLK), jnp.int32)]
            + [pltpu.VMEM((A_BLK, EMBED), jnp.float32) for _ in range(NBUF)]
            + [pltpu.VMEM((8, 8, A_BLK + 1), jnp.float32) for _ in range(NBUF)]
            + [pltpu.SemaphoreType.DMA for _ in range(2 * NBUF)]
        ),
    )
    def lookup(idx_hbm, table_hbm, out_hbm, idx_v, *scratch):
        gbufs = scratch[:NBUF]
        tbufs = scratch[NBUF : 2 * NBUF]
        gsem = scratch[2 * NBUF : 3 * NBUF]
        ssem = scratch[3 * NBUF :]

        wid = lax.axis_index("s") * NC + lax.axis_index("c")
        a0 = wid * A_BLK
        pltpu.sync_copy(idx_hbm.at[:, pl.ds(a0, A_BLK)], idx_v)

        iota = lax.iota(jnp.int32, LANES)
        cvec = [iota + LANES * j for j in range(EMBED // LANES)]
        tr_idx = [lax.shift_right_logical(cv, 3) for cv in cvec]
        r_idx = [lax.bitwise_and(cv, 7) for cv in cvec]

        def gather_start(c, b):
            pltpu.async_copy(table_hbm.at[idx_v.at[c]], gbufs[b], gsem[b])

        def gather_wait(c, b):
            pltpu.make_async_copy(
                table_hbm.at[idx_v.at[c]], gbufs[b], gsem[b]
            ).wait()

        def out_slice(c):
            return out_hbm.at[c, :, wid]

        def tbuf_slice(b):
            return tbufs[b].at[:, :, pl.ds(0, A_BLK)]

        def store_start(c, b):
            pltpu.async_copy(tbuf_slice(b), out_slice(c), ssem[b])

        def store_wait(c, b):
            pltpu.make_async_copy(tbuf_slice(b), out_slice(c), ssem[b]).wait()

        for c in range(PREF):
            gather_start(c, c)

        def outer(i, carry):
            for b in range(NBUF):
                c = i * NBUF + b
                gather_wait(c, b)

                gbuf, tbuf = gbufs[b], tbufs[b]

                @plsc.parallel_loop(0, A_BLK, unroll=4)
                def _transpose_scale(i):
                    col = jnp.full((LANES,), i, jnp.int32)
                    for j in range(EMBED // LANES):
                        v = gbuf[i, pl.ds(j * LANES, LANES)] * SCALE
                        plsc.store_scatter(
                            tbuf, [tr_idx[j], r_idx[j], col], v
                        )

                store_start(c, b)

                bt = (b + PREF) % NBUF
                ct = c + PREF

                @pl.when(ct < n_chunk)
                def _prefetch():
                    @pl.when(c >= NBUF - PREF)
                    def _drain():
                        store_wait(ct - NBUF, bt)

                    gather_start(ct, bt)

            return carry

        lax.fori_loop(0, n_chunk // NBUF, outer, 0)

        for b in range(NBUF):
            store_wait(n_chunk - NBUF + b, b)

    return lookup


def kernel(inputs, table):
    b0, b1 = inputs.shape
    idx_t = jnp.transpose(inputs).astype(jnp.int32)
    out5 = _build_lookup(b0, b1)(idx_t, table)
    return out5.transpose(2, 4, 0, 1, 3).reshape(b0, b1, EMBED)
